# baseline (speedup 1.0000x reference)
import functools

import jax
import jax.numpy as jnp
from jax.experimental import pallas as pl
from jax.experimental.pallas import tpu as pltpu

TOP_K = 2
CAPACITY_FACTOR = 1.25
ENTROPY_THRESHOLD = 1.0
EPS = 1e-8

LB = 256
DC = 512


def _routing(logits, t, ew, cw, uw, run_col, capacity):
    E = logits.shape[0]
    logits = logits / t
    m = jnp.max(logits, axis=0, keepdims=True)
    ex = jnp.exp(logits - m)
    p = ex / jnp.sum(ex, axis=0, keepdims=True)

    ent = -jnp.sum(p * jnp.log(p + EPS), axis=0, keepdims=True)
    mean = jnp.mean(p, axis=0, keepdims=True)
    var = jnp.sum((p - mean) ** 2, axis=0, keepdims=True) / (E - 1)
    conf = 1.0 / (var + EPS)
    ent_norm = jax.nn.sigmoid(ent / ENTROPY_THRESHOLD)
    af = jax.nn.sigmoid(ew * ent_norm + cw * conf + uw * var)

    mp = p * (1.0 + af)
    mp = mp / jnp.sum(mp, axis=0, keepdims=True)

    e_iota = jax.lax.broadcasted_iota(jnp.int32, mp.shape, 0)
    m1 = jnp.max(mp, axis=0, keepdims=True)
    i1 = jnp.min(jnp.where(mp == m1, e_iota, E), axis=0, keepdims=True)
    mask1 = (e_iota == i1)
    mp2 = jnp.where(mask1, -jnp.inf, mp)
    m2 = jnp.max(mp2, axis=0, keepdims=True)
    i2 = jnp.min(jnp.where(mp2 == m2, e_iota, E), axis=0, keepdims=True)
    mask2 = (e_iota == i2)
    wn = jnp.clip(m1 + m2, 1e-9, None)
    disp = mask1.astype(jnp.float32) * (m1 / wn) \
         + mask2.astype(jnp.float32) * (m2 / wn)

    assign = (disp > 0).astype(jnp.float32)
    n = assign.shape[1]
    r = jax.lax.broadcasted_iota(jnp.int32, (n, n), 0)
    c = jax.lax.broadcasted_iota(jnp.int32, (n, n), 1)
    triu = (r <= c).astype(jnp.float32)
    csum = jax.lax.dot_general(assign, triu, (((1,), (0,)), ((), ())),
                               preferred_element_type=jnp.float32)
    positions = run_col + csum - 1.0
    keep = (positions < float(capacity)).astype(jnp.float32)
    disp = disp * keep
    return disp, p, ent, jnp.sum(assign, axis=1, keepdims=True)


def _fused_kernel(params_ref, x_ref, gw_ref, gb_ref, w_ref, b_ref,
                  out_ref, counts_ref, gates_ref, ents_ref,
                  disp_scr, pooled_scr, pooled_t_scr, eo_scr, invc_scr, run_scr,
                  *, capacity, B, E, n1, n2, n3, num_dc):
    i = pl.program_id(0)

    @pl.when(i == 0)
    def _init():
        run_scr[...] = jnp.zeros_like(run_scr)
        pooled_scr[...] = jnp.zeros_like(pooled_scr)
        counts_ref[...] = jnp.zeros_like(counts_ref)
        gates_ref[...] = jnp.zeros_like(gates_ref)
        ents_ref[...] = jnp.zeros_like(ents_ref)

    @pl.when(i < n1)
    def _phase1():
        t = params_ref[0]
        ew = params_ref[1]
        cw = params_ref[2]
        uw = params_ref[3]
        gw = gw_ref[...]
        gb = gb_ref[...]
        for b in range(B):
            xb = x_ref[b]
            logits = jax.lax.dot_general(gw, xb, (((1,), (1,)), ((), ())),
                                         preferred_element_type=jnp.float32)
            disp, p, ent, nass = _routing(logits + gb, t, ew, cw, uw,
                                          run_scr[b], capacity)
            run_scr[b] += nass
            disp_scr[b, :, pl.ds(i * LB, LB)] = disp
            pooled_scr[b] += jax.lax.dot_general(
                disp, xb, (((1,), (0,)), ((), ())),
                preferred_element_type=jnp.float32)
            counts_ref[b] += jnp.sum(disp, axis=1, keepdims=True)
            gates_ref[b] += jnp.sum(p, axis=1, keepdims=True)
            ents_ref[b] += jnp.broadcast_to(jnp.sum(ent, keepdims=True),
                                            ents_ref[b].shape)

    @pl.when(i == n1)
    def _transition():
        pooled_t_scr[...] = jnp.swapaxes(pooled_scr[...], 0, 1)
        cnt = counts_ref[...][:, :, 0]
        invc_scr[...] = (1.0 / jnp.clip(cnt.T, 1.0, None))[:, :, None]

    @pl.when((i >= n1) & (i < n1 + n2))
    def _phase2():
        j = i - n1
        e = j // num_dc
        dc = j % num_dc
        acc = jax.lax.dot_general(pooled_t_scr[e], w_ref[0],
                                  (((1,), (1,)), ((), ())),
                                  preferred_element_type=jnp.float32)
        eo_scr[e, :, pl.ds(dc * DC, DC)] = acc * invc_scr[e] + b_ref[0]

    @pl.when(i >= n1 + n2)
    def _phase3():
        l3 = i - n1 - n2
        for b in range(B):
            dispb = disp_scr[b, :, pl.ds(l3 * LB, LB)]
            out_ref[b] = jax.lax.dot_general(
                dispb, eo_scr[:, b, :], (((0,), (0,)), ((), ())),
                preferred_element_type=jnp.float32)


def kernel(x, gate_W, gate_b, expert_W, expert_b, temperature,
           entropy_weight, confidence_weight, uncertainty_weight):
    B, L, D = x.shape
    E = gate_W.shape[0]
    capacity = int(CAPACITY_FACTOR * (B * L / max(1, E)) + 0.9999)
    n1 = L // LB
    num_dc = D // DC
    n2 = E * num_dc
    n3 = L // LB
    nsteps = n1 + n2 + n3

    params = jnp.concatenate([temperature, entropy_weight,
                              confidence_weight, uncertainty_weight])
    gb2 = gate_b.reshape(E, 1)
    eb3 = expert_b[:, None, :]

    def x_idx(i):
        return (0, jnp.minimum(i, n1 - 1), 0)

    def w_idx(i):
        j = jnp.clip(i - n1, 0, n2 - 1)
        return (j // num_dc, j % num_dc, 0)

    def eb_idx(i):
        j = jnp.clip(i - n1, 0, n2 - 1)
        return (j // num_dc, 0, j % num_dc)

    def out_idx(i):
        return (0, jnp.clip(i - n1 - n2, 0, n3 - 1), 0)

    out, counts, gates, ents = pl.pallas_call(
        functools.partial(_fused_kernel, capacity=capacity, B=B, E=E,
                          n1=n1, n2=n2, n3=n3, num_dc=num_dc),
        grid=(nsteps,),
        in_specs=[
            pl.BlockSpec(memory_space=pltpu.SMEM),
            pl.BlockSpec((B, LB, D), x_idx),
            pl.BlockSpec((E, D), lambda i: (0, 0)),
            pl.BlockSpec((E, 1), lambda i: (0, 0)),
            pl.BlockSpec((1, DC, D), w_idx),
            pl.BlockSpec((1, 1, DC), eb_idx),
        ],
        out_specs=[
            pl.BlockSpec((B, LB, D), out_idx),
            pl.BlockSpec((B, E, 1), lambda i: (0, 0, 0)),
            pl.BlockSpec((B, E, 1), lambda i: (0, 0, 0)),
            pl.BlockSpec((B, E, 1), lambda i: (0, 0, 0)),
        ],
        out_shape=[
            jax.ShapeDtypeStruct((B, L, D), jnp.float32),
            jax.ShapeDtypeStruct((B, E, 1), jnp.float32),
            jax.ShapeDtypeStruct((B, E, 1), jnp.float32),
            jax.ShapeDtypeStruct((B, E, 1), jnp.float32),
        ],
        scratch_shapes=[
            pltpu.VMEM((B, E, L), jnp.float32),
            pltpu.VMEM((B, E, D), jnp.float32),
            pltpu.VMEM((E, B, D), jnp.float32),
            pltpu.VMEM((E, B, D), jnp.float32),
            pltpu.VMEM((E, B, 1), jnp.float32),
            pltpu.VMEM((B, E, 1), jnp.float32),
        ],
    )(params, x, gate_W, gb2, expert_W, eb3)

    counts2 = counts[:, :, 0]
    util = jnp.sum(counts2, axis=0) / (B * L)
    diversity_loss = -jnp.var(util, ddof=1) * 0.01
    mean_gate = gates[:, :, 0] / L
    aux_loss = jnp.var(mean_gate) * E + diversity_loss
    avg_ent = jnp.sum(ents[:, 0, 0]) / (B * L)
    aux_loss = aux_loss + (avg_ent - ENTROPY_THRESHOLD) ** 2 * 0.01
    return (out, aux_loss)


# speedup vs baseline: 1.0607x; 1.0607x over previous
import functools

import jax
import jax.numpy as jnp
from jax.experimental import pallas as pl
from jax.experimental.pallas import tpu as pltpu

TOP_K = 2
CAPACITY_FACTOR = 1.25
ENTROPY_THRESHOLD = 1.0
EPS = 1e-8

LB = 512
DC = 512
LB3 = 512


def _phase1_kernel(params_ref, x_ref, gw_ref, gb_ref,
                   disp_ref, pooled_ref, counts_ref, gates_ref, ents_ref,
                   run_ref, *, capacity):
    lb = pl.program_id(1)

    @pl.when(lb == 0)
    def _init():
        run_ref[...] = jnp.zeros_like(run_ref)
        pooled_ref[...] = jnp.zeros_like(pooled_ref)
        counts_ref[...] = jnp.zeros_like(counts_ref)
        gates_ref[...] = jnp.zeros_like(gates_ref)
        ents_ref[...] = jnp.zeros_like(ents_ref)

    xb = x_ref[0]
    gw = gw_ref[...]
    E = gw.shape[0]
    t = params_ref[0]
    ew = params_ref[1]
    cw = params_ref[2]
    uw = params_ref[3]

    logits = jax.lax.dot_general(gw, xb, (((1,), (1,)), ((), ())),
                                 preferred_element_type=jnp.float32)
    logits = (logits + gb_ref[...]) / t

    m = jnp.max(logits, axis=0, keepdims=True)
    ex = jnp.exp(logits - m)
    p = ex / jnp.sum(ex, axis=0, keepdims=True)

    ent = -jnp.sum(p * jnp.log(p + EPS), axis=0, keepdims=True)
    mean = jnp.mean(p, axis=0, keepdims=True)
    var = jnp.sum((p - mean) ** 2, axis=0, keepdims=True) / (E - 1)
    conf = 1.0 / (var + EPS)
    ent_norm = jax.nn.sigmoid(ent / ENTROPY_THRESHOLD)
    af = jax.nn.sigmoid(ew * ent_norm + cw * conf + uw * var)

    mp = p * (1.0 + af)
    mp = mp / jnp.sum(mp, axis=0, keepdims=True)

    e_iota = jax.lax.broadcasted_iota(jnp.int32, mp.shape, 0)
    m1 = jnp.max(mp, axis=0, keepdims=True)
    i1 = jnp.min(jnp.where(mp == m1, e_iota, E), axis=0, keepdims=True)
    mask1 = (e_iota == i1)
    mp2 = jnp.where(mask1, -jnp.inf, mp)
    m2 = jnp.max(mp2, axis=0, keepdims=True)
    i2 = jnp.min(jnp.where(mp2 == m2, e_iota, E), axis=0, keepdims=True)
    mask2 = (e_iota == i2)
    wn = jnp.clip(m1 + m2, 1e-9, None)
    disp = mask1.astype(jnp.float32) * (m1 / wn) \
         + mask2.astype(jnp.float32) * (m2 / wn)

    assign = (disp > 0).astype(jnp.float32)
    n = assign.shape[1]
    r = jax.lax.broadcasted_iota(jnp.int32, (n, n), 0)
    c = jax.lax.broadcasted_iota(jnp.int32, (n, n), 1)
    triu = (r <= c).astype(jnp.float32)
    csum = jax.lax.dot_general(assign, triu, (((1,), (0,)), ((), ())),
                               preferred_element_type=jnp.float32)
    positions = run_ref[...] + csum - 1.0
    keep = (positions < float(capacity)).astype(jnp.float32)
    disp = disp * keep
    run_ref[...] += jnp.sum(assign, axis=1, keepdims=True)

    disp_ref[0] = disp
    pooled_ref[0] += jax.lax.dot_general(disp, xb, (((1,), (0,)), ((), ())),
                                         preferred_element_type=jnp.float32)
    counts_ref[0] += jnp.sum(disp, axis=1, keepdims=True)
    gates_ref[0] += jnp.sum(p, axis=1, keepdims=True)
    ents_ref[0] += jnp.broadcast_to(jnp.sum(ent, keepdims=True), ents_ref[0].shape)


def _phase23_kernel(pooled_ref, counts_ref, w_ref, b_ref, disp_ref,
                    out_ref, pooled_t_scr, eo_scr, invc_scr,
                    *, B, n2, num_dc):
    i = pl.program_id(0)

    @pl.when(i == 0)
    def _transition():
        pooled_t_scr[...] = jnp.swapaxes(pooled_ref[...], 0, 1)
        cnt = counts_ref[...][:, :, 0]
        invc_scr[...] = (1.0 / jnp.clip(cnt.T, 1.0, None))[:, :, None]

    @pl.when(i < n2)
    def _phase2():
        e = i // num_dc
        dc = i % num_dc
        acc = jax.lax.dot_general(pooled_t_scr[e], w_ref[0],
                                  (((1,), (1,)), ((), ())),
                                  preferred_element_type=jnp.float32)
        eo_scr[e, :, pl.ds(dc * DC, DC)] = acc * invc_scr[e] + b_ref[0]

    @pl.when(i >= n2)
    def _phase3():
        for b in range(B):
            out_ref[b] = jax.lax.dot_general(
                disp_ref[b], eo_scr[:, b, :], (((0,), (0,)), ((), ())),
                preferred_element_type=jnp.float32)


def kernel(x, gate_W, gate_b, expert_W, expert_b, temperature,
           entropy_weight, confidence_weight, uncertainty_weight):
    B, L, D = x.shape
    E = gate_W.shape[0]
    capacity = int(CAPACITY_FACTOR * (B * L / max(1, E)) + 0.9999)
    num_lb = L // LB

    params = jnp.concatenate([temperature, entropy_weight,
                              confidence_weight, uncertainty_weight])
    gb2 = gate_b.reshape(E, 1)
    eb3 = expert_b[:, None, :]

    disp, pooled, counts, gates, ents = pl.pallas_call(
        functools.partial(_phase1_kernel, capacity=capacity),
        grid=(B, num_lb),
        in_specs=[
            pl.BlockSpec(memory_space=pltpu.SMEM),
            pl.BlockSpec((1, LB, D), lambda b, l: (b, l, 0)),
            pl.BlockSpec((E, D), lambda b, l: (0, 0)),
            pl.BlockSpec((E, 1), lambda b, l: (0, 0)),
        ],
        out_specs=[
            pl.BlockSpec((1, E, LB), lambda b, l: (b, 0, l)),
            pl.BlockSpec((1, E, D), lambda b, l: (b, 0, 0)),
            pl.BlockSpec((1, E, 1), lambda b, l: (b, 0, 0)),
            pl.BlockSpec((1, E, 1), lambda b, l: (b, 0, 0)),
            pl.BlockSpec((1, E, 1), lambda b, l: (b, 0, 0)),
        ],
        out_shape=[
            jax.ShapeDtypeStruct((B, E, L), jnp.float32),
            jax.ShapeDtypeStruct((B, E, D), jnp.float32),
            jax.ShapeDtypeStruct((B, E, 1), jnp.float32),
            jax.ShapeDtypeStruct((B, E, 1), jnp.float32),
            jax.ShapeDtypeStruct((B, E, 1), jnp.float32),
        ],
        scratch_shapes=[pltpu.VMEM((E, 1), jnp.float32)],
    )(params, x, gate_W, gb2)

    num_dc = D // DC
    n2 = E * num_dc
    n3 = L // LB3

    def w_idx(i):
        j = jnp.clip(i, 0, n2 - 1)
        return (j // num_dc, j % num_dc, 0)

    def eb_idx(i):
        j = jnp.clip(i, 0, n2 - 1)
        return (j // num_dc, 0, j % num_dc)

    def disp_idx(i):
        return (0, 0, jnp.clip(i - n2, 0, n3 - 1))

    def out_idx(i):
        return (0, jnp.clip(i - n2, 0, n3 - 1), 0)

    out = pl.pallas_call(
        functools.partial(_phase23_kernel, B=B, n2=n2, num_dc=num_dc),
        grid=(n2 + n3,),
        in_specs=[
            pl.BlockSpec((B, E, D), lambda i: (0, 0, 0)),
            pl.BlockSpec((B, E, 1), lambda i: (0, 0, 0)),
            pl.BlockSpec((1, DC, D), w_idx),
            pl.BlockSpec((1, 1, DC), eb_idx),
            pl.BlockSpec((B, E, LB3), disp_idx),
        ],
        out_specs=pl.BlockSpec((B, LB3, D), out_idx),
        out_shape=jax.ShapeDtypeStruct((B, L, D), jnp.float32),
        scratch_shapes=[
            pltpu.VMEM((E, B, D), jnp.float32),
            pltpu.VMEM((E, B, D), jnp.float32),
            pltpu.VMEM((E, B, 1), jnp.float32),
        ],
    )(pooled, counts, expert_W, eb3, disp)

    counts2 = counts[:, :, 0]
    util = jnp.sum(counts2, axis=0) / (B * L)
    diversity_loss = -jnp.var(util, ddof=1) * 0.01
    mean_gate = gates[:, :, 0] / L
    aux_loss = jnp.var(mean_gate) * E + diversity_loss
    avg_ent = jnp.sum(ents[:, 0, 0]) / (B * L)
    aux_loss = aux_loss + (avg_ent - ENTROPY_THRESHOLD) ** 2 * 0.01
    return (out, aux_loss)


# speedup vs baseline: 1.1215x; 1.0573x over previous
import functools

import jax
import jax.numpy as jnp
from jax.experimental import pallas as pl
from jax.experimental.pallas import tpu as pltpu

TOP_K = 2
CAPACITY_FACTOR = 1.25
ENTROPY_THRESHOLD = 1.0
EPS = 1e-8

LB = 512
DC = 2048
LB3 = 256


def _phase1_kernel(params_ref, x_ref, gw_ref, gb_ref,
                   disp_ref, pooled_ref, counts_ref, gates_ref, ents_ref,
                   run_ref, *, capacity):
    lb = pl.program_id(1)

    @pl.when(lb == 0)
    def _init():
        run_ref[...] = jnp.zeros_like(run_ref)
        pooled_ref[...] = jnp.zeros_like(pooled_ref)
        counts_ref[...] = jnp.zeros_like(counts_ref)
        gates_ref[...] = jnp.zeros_like(gates_ref)
        ents_ref[...] = jnp.zeros_like(ents_ref)

    xb = x_ref[0]
    gw = gw_ref[...]
    E = gw.shape[0]
    t = params_ref[0]
    ew = params_ref[1]
    cw = params_ref[2]
    uw = params_ref[3]

    logits = jax.lax.dot_general(gw, xb, (((1,), (1,)), ((), ())),
                                 preferred_element_type=jnp.float32)
    logits = (logits + gb_ref[...]) / t

    m = jnp.max(logits, axis=0, keepdims=True)
    ex = jnp.exp(logits - m)
    p = ex / jnp.sum(ex, axis=0, keepdims=True)

    ent = -jnp.sum(p * jnp.log(p + EPS), axis=0, keepdims=True)
    mean = jnp.mean(p, axis=0, keepdims=True)
    var = jnp.sum((p - mean) ** 2, axis=0, keepdims=True) / (E - 1)
    conf = 1.0 / (var + EPS)
    ent_norm = jax.nn.sigmoid(ent / ENTROPY_THRESHOLD)
    af = jax.nn.sigmoid(ew * ent_norm + cw * conf + uw * var)

    mp = p * (1.0 + af)
    mp = mp / jnp.sum(mp, axis=0, keepdims=True)

    e_iota = jax.lax.broadcasted_iota(jnp.int32, mp.shape, 0)
    m1 = jnp.max(mp, axis=0, keepdims=True)
    i1 = jnp.min(jnp.where(mp == m1, e_iota, E), axis=0, keepdims=True)
    mask1 = (e_iota == i1)
    mp2 = jnp.where(mask1, -jnp.inf, mp)
    m2 = jnp.max(mp2, axis=0, keepdims=True)
    i2 = jnp.min(jnp.where(mp2 == m2, e_iota, E), axis=0, keepdims=True)
    mask2 = (e_iota == i2)
    wn = jnp.clip(m1 + m2, 1e-9, None)
    disp = mask1.astype(jnp.float32) * (m1 / wn) \
         + mask2.astype(jnp.float32) * (m2 / wn)

    assign = (disp > 0).astype(jnp.float32)
    n = assign.shape[1]
    r = jax.lax.broadcasted_iota(jnp.int32, (n, n), 0)
    c = jax.lax.broadcasted_iota(jnp.int32, (n, n), 1)
    triu = (r <= c).astype(jnp.float32)
    csum = jax.lax.dot_general(assign, triu, (((1,), (0,)), ((), ())),
                               preferred_element_type=jnp.float32)
    positions = run_ref[...] + csum - 1.0
    keep = (positions < float(capacity)).astype(jnp.float32)
    disp = disp * keep
    run_ref[...] += jnp.sum(assign, axis=1, keepdims=True)

    disp_ref[0] = disp
    pooled_ref[0] += jax.lax.dot_general(disp, xb, (((1,), (0,)), ((), ())),
                                         preferred_element_type=jnp.float32)
    counts_ref[0] += jnp.sum(disp, axis=1, keepdims=True)
    gates_ref[0] += jnp.sum(p, axis=1, keepdims=True)
    ents_ref[0] += jnp.broadcast_to(jnp.sum(ent, keepdims=True), ents_ref[0].shape)


def _phase23_kernel(pooled_ref, counts_ref, w_ref, b_ref, disp_ref,
                    out_ref, pooled_t_scr, eo_scr, invc_scr,
                    *, B, n2, num_dc):
    i = pl.program_id(0)

    @pl.when(i == 0)
    def _transition():
        pooled_t_scr[...] = jnp.swapaxes(pooled_ref[...], 0, 1)
        cnt = counts_ref[...][:, :, 0]
        invc_scr[...] = (1.0 / jnp.clip(cnt.T, 1.0, None))[:, :, None]

    @pl.when(i < n2)
    def _phase2():
        e = i // num_dc
        dc = i % num_dc
        acc = jax.lax.dot_general(pooled_t_scr[e], w_ref[0],
                                  (((1,), (1,)), ((), ())),
                                  preferred_element_type=jnp.float32)
        eo_scr[e, :, pl.ds(dc * DC, DC)] = acc * invc_scr[e] + b_ref[0]

    @pl.when(i >= n2)
    def _phase3():
        for b in range(B):
            out_ref[b] = jax.lax.dot_general(
                disp_ref[b], eo_scr[:, b, :], (((0,), (0,)), ((), ())),
                preferred_element_type=jnp.float32)


def kernel(x, gate_W, gate_b, expert_W, expert_b, temperature,
           entropy_weight, confidence_weight, uncertainty_weight):
    B, L, D = x.shape
    E = gate_W.shape[0]
    capacity = int(CAPACITY_FACTOR * (B * L / max(1, E)) + 0.9999)
    num_lb = L // LB

    params = jnp.concatenate([temperature, entropy_weight,
                              confidence_weight, uncertainty_weight])
    gb2 = gate_b.reshape(E, 1)
    eb3 = expert_b[:, None, :]

    disp, pooled, counts, gates, ents = pl.pallas_call(
        functools.partial(_phase1_kernel, capacity=capacity),
        grid=(B, num_lb),
        in_specs=[
            pl.BlockSpec(memory_space=pltpu.SMEM),
            pl.BlockSpec((1, LB, D), lambda b, l: (b, l, 0)),
            pl.BlockSpec((E, D), lambda b, l: (0, 0)),
            pl.BlockSpec((E, 1), lambda b, l: (0, 0)),
        ],
        out_specs=[
            pl.BlockSpec((1, E, LB), lambda b, l: (b, 0, l)),
            pl.BlockSpec((1, E, D), lambda b, l: (b, 0, 0)),
            pl.BlockSpec((1, E, 1), lambda b, l: (b, 0, 0)),
            pl.BlockSpec((1, E, 1), lambda b, l: (b, 0, 0)),
            pl.BlockSpec((1, E, 1), lambda b, l: (b, 0, 0)),
        ],
        out_shape=[
            jax.ShapeDtypeStruct((B, E, L), jnp.float32),
            jax.ShapeDtypeStruct((B, E, D), jnp.float32),
            jax.ShapeDtypeStruct((B, E, 1), jnp.float32),
            jax.ShapeDtypeStruct((B, E, 1), jnp.float32),
            jax.ShapeDtypeStruct((B, E, 1), jnp.float32),
        ],
        scratch_shapes=[pltpu.VMEM((E, 1), jnp.float32)],
    )(params, x, gate_W, gb2)

    num_dc = D // DC
    n2 = E * num_dc
    n3 = L // LB3

    def w_idx(i):
        j = jnp.clip(i, 0, n2 - 1)
        return (j // num_dc, j % num_dc, 0)

    def eb_idx(i):
        j = jnp.clip(i, 0, n2 - 1)
        return (j // num_dc, 0, j % num_dc)

    def disp_idx(i):
        return (0, 0, jnp.clip(i - n2, 0, n3 - 1))

    def out_idx(i):
        return (0, jnp.clip(i - n2, 0, n3 - 1), 0)

    out = pl.pallas_call(
        functools.partial(_phase23_kernel, B=B, n2=n2, num_dc=num_dc),
        grid=(n2 + n3,),
        in_specs=[
            pl.BlockSpec((B, E, D), lambda i: (0, 0, 0)),
            pl.BlockSpec((B, E, 1), lambda i: (0, 0, 0)),
            pl.BlockSpec((1, DC, D), w_idx),
            pl.BlockSpec((1, 1, DC), eb_idx),
            pl.BlockSpec((B, E, LB3), disp_idx),
        ],
        out_specs=pl.BlockSpec((B, LB3, D), out_idx),
        out_shape=jax.ShapeDtypeStruct((B, L, D), jnp.float32),
        scratch_shapes=[
            pltpu.VMEM((E, B, D), jnp.float32),
            pltpu.VMEM((E, B, D), jnp.float32),
            pltpu.VMEM((E, B, 1), jnp.float32),
        ],
    )(pooled, counts, expert_W, eb3, disp)

    counts2 = counts[:, :, 0]
    util = jnp.sum(counts2, axis=0) / (B * L)
    diversity_loss = -jnp.var(util, ddof=1) * 0.01
    mean_gate = gates[:, :, 0] / L
    aux_loss = jnp.var(mean_gate) * E + diversity_loss
    avg_ent = jnp.sum(ents[:, 0, 0]) / (B * L)
    aux_loss = aux_loss + (avg_ent - ENTROPY_THRESHOLD) ** 2 * 0.01
    return (out, aux_loss)


# speedup vs baseline: 1.1744x; 1.0472x over previous
import functools

import jax
import jax.numpy as jnp
from jax.experimental import pallas as pl
from jax.experimental.pallas import tpu as pltpu

TOP_K = 2
CAPACITY_FACTOR = 1.25
ENTROPY_THRESHOLD = 1.0
EPS = 1e-8

LB = 1024
DC = 2048
LB3 = 256


def _phase1_kernel(params_ref, x_ref, gw_ref, gb_ref,
                   disp_ref, pooled_ref, counts_ref, gates_ref, ents_ref,
                   run_ref, *, capacity):
    lb = pl.program_id(1)

    @pl.when(lb == 0)
    def _init():
        run_ref[...] = jnp.zeros_like(run_ref)
        pooled_ref[...] = jnp.zeros_like(pooled_ref)
        counts_ref[...] = jnp.zeros_like(counts_ref)
        gates_ref[...] = jnp.zeros_like(gates_ref)
        ents_ref[...] = jnp.zeros_like(ents_ref)

    xb = x_ref[0]
    gw = gw_ref[...]
    E = gw.shape[0]
    t = params_ref[0]
    ew = params_ref[1]
    cw = params_ref[2]
    uw = params_ref[3]

    logits = jax.lax.dot_general(gw, xb, (((1,), (1,)), ((), ())),
                                 preferred_element_type=jnp.float32)
    logits = (logits + gb_ref[...]) / t

    m = jnp.max(logits, axis=0, keepdims=True)
    ex = jnp.exp(logits - m)
    p = ex / jnp.sum(ex, axis=0, keepdims=True)

    ent = -jnp.sum(p * jnp.log(p + EPS), axis=0, keepdims=True)
    mean = jnp.mean(p, axis=0, keepdims=True)
    var = jnp.sum((p - mean) ** 2, axis=0, keepdims=True) / (E - 1)
    conf = 1.0 / (var + EPS)
    ent_norm = jax.nn.sigmoid(ent / ENTROPY_THRESHOLD)
    af = jax.nn.sigmoid(ew * ent_norm + cw * conf + uw * var)

    mp = p * (1.0 + af)
    mp = mp / jnp.sum(mp, axis=0, keepdims=True)

    e_iota = jax.lax.broadcasted_iota(jnp.int32, mp.shape, 0)
    m1 = jnp.max(mp, axis=0, keepdims=True)
    i1 = jnp.min(jnp.where(mp == m1, e_iota, E), axis=0, keepdims=True)
    mask1 = (e_iota == i1)
    mp2 = jnp.where(mask1, -jnp.inf, mp)
    m2 = jnp.max(mp2, axis=0, keepdims=True)
    i2 = jnp.min(jnp.where(mp2 == m2, e_iota, E), axis=0, keepdims=True)
    mask2 = (e_iota == i2)
    wn = jnp.clip(m1 + m2, 1e-9, None)
    disp = mask1.astype(jnp.float32) * (m1 / wn) \
         + mask2.astype(jnp.float32) * (m2 / wn)

    assign = (disp > 0).astype(jnp.float32)
    n = assign.shape[1]
    r = jax.lax.broadcasted_iota(jnp.int32, (n, n), 0)
    c = jax.lax.broadcasted_iota(jnp.int32, (n, n), 1)
    triu = (r <= c).astype(jnp.float32)
    csum = jax.lax.dot_general(assign, triu, (((1,), (0,)), ((), ())),
                               preferred_element_type=jnp.float32)
    positions = run_ref[...] + csum - 1.0
    keep = (positions < float(capacity)).astype(jnp.float32)
    disp = disp * keep
    run_ref[...] += jnp.sum(assign, axis=1, keepdims=True)

    disp_ref[0] = disp
    pooled_ref[0] += jax.lax.dot_general(disp, xb, (((1,), (0,)), ((), ())),
                                         preferred_element_type=jnp.float32)
    counts_ref[0] += jnp.sum(disp, axis=1, keepdims=True)
    gates_ref[0] += jnp.sum(p, axis=1, keepdims=True)
    ents_ref[0] += jnp.broadcast_to(jnp.sum(ent, keepdims=True), ents_ref[0].shape)


def _phase23_kernel(pooled_ref, counts_ref, w_ref, b_ref, disp_ref,
                    out_ref, pooled_t_scr, eo_scr, invc_scr,
                    *, B, n2, num_dc):
    i = pl.program_id(0)

    @pl.when(i == 0)
    def _transition():
        pooled_t_scr[...] = jnp.swapaxes(pooled_ref[...], 0, 1)
        cnt = counts_ref[...][:, :, 0]
        invc_scr[...] = (1.0 / jnp.clip(cnt.T, 1.0, None))[:, :, None]

    @pl.when(i < n2)
    def _phase2():
        e = i // num_dc
        dc = i % num_dc
        acc = jax.lax.dot_general(pooled_t_scr[e], w_ref[0],
                                  (((1,), (1,)), ((), ())),
                                  preferred_element_type=jnp.float32)
        eo_scr[e, :, pl.ds(dc * DC, DC)] = acc * invc_scr[e] + b_ref[0]

    @pl.when(i >= n2)
    def _phase3():
        for b in range(B):
            out_ref[b] = jax.lax.dot_general(
                disp_ref[b], eo_scr[:, b, :], (((0,), (0,)), ((), ())),
                preferred_element_type=jnp.float32)


def kernel(x, gate_W, gate_b, expert_W, expert_b, temperature,
           entropy_weight, confidence_weight, uncertainty_weight):
    B, L, D = x.shape
    E = gate_W.shape[0]
    capacity = int(CAPACITY_FACTOR * (B * L / max(1, E)) + 0.9999)
    num_lb = L // LB

    params = jnp.concatenate([temperature, entropy_weight,
                              confidence_weight, uncertainty_weight])
    gb2 = gate_b.reshape(E, 1)
    eb3 = expert_b[:, None, :]

    disp, pooled, counts, gates, ents = pl.pallas_call(
        functools.partial(_phase1_kernel, capacity=capacity),
        grid=(B, num_lb),
        in_specs=[
            pl.BlockSpec(memory_space=pltpu.SMEM),
            pl.BlockSpec((1, LB, D), lambda b, l: (b, l, 0)),
            pl.BlockSpec((E, D), lambda b, l: (0, 0)),
            pl.BlockSpec((E, 1), lambda b, l: (0, 0)),
        ],
        out_specs=[
            pl.BlockSpec((1, E, LB), lambda b, l: (b, 0, l)),
            pl.BlockSpec((1, E, D), lambda b, l: (b, 0, 0)),
            pl.BlockSpec((1, E, 1), lambda b, l: (b, 0, 0)),
            pl.BlockSpec((1, E, 1), lambda b, l: (b, 0, 0)),
            pl.BlockSpec((1, E, 1), lambda b, l: (b, 0, 0)),
        ],
        out_shape=[
            jax.ShapeDtypeStruct((B, E, L), jnp.float32),
            jax.ShapeDtypeStruct((B, E, D), jnp.float32),
            jax.ShapeDtypeStruct((B, E, 1), jnp.float32),
            jax.ShapeDtypeStruct((B, E, 1), jnp.float32),
            jax.ShapeDtypeStruct((B, E, 1), jnp.float32),
        ],
        scratch_shapes=[pltpu.VMEM((E, 1), jnp.float32)],
    )(params, x, gate_W, gb2)

    num_dc = D // DC
    n2 = E * num_dc
    n3 = L // LB3

    def w_idx(i):
        j = jnp.clip(i, 0, n2 - 1)
        return (j // num_dc, j % num_dc, 0)

    def eb_idx(i):
        j = jnp.clip(i, 0, n2 - 1)
        return (j // num_dc, 0, j % num_dc)

    def disp_idx(i):
        return (0, 0, jnp.clip(i - n2, 0, n3 - 1))

    def out_idx(i):
        return (0, jnp.clip(i - n2, 0, n3 - 1), 0)

    out = pl.pallas_call(
        functools.partial(_phase23_kernel, B=B, n2=n2, num_dc=num_dc),
        grid=(n2 + n3,),
        in_specs=[
            pl.BlockSpec((B, E, D), lambda i: (0, 0, 0)),
            pl.BlockSpec((B, E, 1), lambda i: (0, 0, 0)),
            pl.BlockSpec((1, DC, D), w_idx),
            pl.BlockSpec((1, 1, DC), eb_idx),
            pl.BlockSpec((B, E, LB3), disp_idx),
        ],
        out_specs=pl.BlockSpec((B, LB3, D), out_idx),
        out_shape=jax.ShapeDtypeStruct((B, L, D), jnp.float32),
        scratch_shapes=[
            pltpu.VMEM((E, B, D), jnp.float32),
            pltpu.VMEM((E, B, D), jnp.float32),
            pltpu.VMEM((E, B, 1), jnp.float32),
        ],
    )(pooled, counts, expert_W, eb3, disp)

    counts2 = counts[:, :, 0]
    util = jnp.sum(counts2, axis=0) / (B * L)
    diversity_loss = -jnp.var(util, ddof=1) * 0.01
    mean_gate = gates[:, :, 0] / L
    aux_loss = jnp.var(mean_gate) * E + diversity_loss
    avg_ent = jnp.sum(ents[:, 0, 0]) / (B * L)
    aux_loss = aux_loss + (avg_ent - ENTROPY_THRESHOLD) ** 2 * 0.01
    return (out, aux_loss)
